# baseline (speedup 1.0000x reference)
import functools

import numpy as np
import jax
import jax.numpy as jnp
from jax import lax
from jax.experimental import pallas as pl
from jax.experimental.pallas import tpu as pltpu
from jax.experimental.pallas import tpu_sc as plsc

N = 10000
E = 160000
HID = 128
H = 16
HO = 8
L = 2

NC = 2
NS = 16
CH = 512
EPT = 10240
EPAD = NS * EPT
NCHUNK = EPT // CH
ROWS_PT = 632
NPAD = NS * ROWS_PT

PERM = np.array([(j % H) * HO + j // H for j in range(HID)], np.int32)

_f32 = jnp.float32


def _embed_body(x_ref, w0_ref, g_ref, b_ref, w1_ref, o_ref):
    h = jnp.dot(x_ref[...], w0_ref[...], preferred_element_type=_f32)
    m = jnp.mean(h, axis=0, keepdims=True)
    v = jnp.mean((h - m) ** 2, axis=0, keepdims=True)
    hn = g_ref[...] * (h - m) / jnp.sqrt(v + 1e-5) + b_ref[...]
    o_ref[...] = jnp.dot(jnp.maximum(hn, 0.0), w1_ref[...],
                         preferred_element_type=_f32)


def _tc_embed(x, w0, g, b, w1):
    return pl.pallas_call(
        _embed_body,
        out_shape=jax.ShapeDtypeStruct((N, HID), _f32),
    )(x, w0, g, b, w1)


BLK = 2000


def _proj_pair(a, b_, W1, Ml1, Wr1, W2, Ml2, Wr2,
               fs1o, el1o, er1o, fs2o, el2o, er2o):
    fs1 = jnp.dot(a, W1, preferred_element_type=_f32)
    fs1o[0] = fs1[:, 0:64]
    fs1o[1] = fs1[:, 64:128]
    el1 = jnp.dot(fs1, Ml1, preferred_element_type=_f32)
    el1o[0] = el1
    el1o[1] = el1
    er1o[...] = jnp.dot(b_, Wr1, preferred_element_type=_f32)
    fs2 = jnp.dot(b_, W2, preferred_element_type=_f32)
    fs2o[0] = fs2[:, 0:64]
    fs2o[1] = fs2[:, 64:128]
    el2 = jnp.dot(fs2, Ml2, preferred_element_type=_f32)
    el2o[0] = el2
    el2o[1] = el2
    er2o[...] = jnp.dot(a, Wr2, preferred_element_type=_f32)


def _proj_body(h1a_ref, h1b_ref, W1_ref, Ml1_ref, Wr1_ref,
               W2_ref, Ml2_ref, Wr2_ref,
               fs1o, el1o, er1o, fs2o, el2o, er2o):
    _proj_pair(h1a_ref[...], h1b_ref[...], W1_ref[...], Ml1_ref[...],
               Wr1_ref[...], W2_ref[...], Ml2_ref[...], Wr2_ref[...],
               fs1o, el1o, er1o, fs2o, el2o, er2o)


_ROWS = pl.BlockSpec((BLK, HID), lambda i: (i, 0))
_WFULL = pl.BlockSpec((HID, HID), lambda i: (0, 0))
_WSMALL = pl.BlockSpec((HID, H), lambda i: (0, 0))
_BIAS = pl.BlockSpec((1, HID), lambda i: (0, 0))
_FSBLK = pl.BlockSpec((2, BLK, 64), lambda i: (0, i, 0))
_ELBLK = pl.BlockSpec((BLK, H), lambda i: (i, 0))
_EL2BLK = pl.BlockSpec((2, BLK, H), lambda i: (0, i, 0))


def _proj_outs():
    return [
        jax.ShapeDtypeStruct((2, N, 64), _f32),
        jax.ShapeDtypeStruct((2, N, H), _f32),
        jax.ShapeDtypeStruct((N, H), _f32),
        jax.ShapeDtypeStruct((2, N, 64), _f32),
        jax.ShapeDtypeStruct((2, N, H), _f32),
        jax.ShapeDtypeStruct((N, H), _f32),
    ]


_PROJ_OUT_SPECS = [_FSBLK, _EL2BLK, _ELBLK, _FSBLK, _EL2BLK, _ELBLK]


def _tc_proj0(h1a, h1b, W1, Ml1, Wr1, W2, Ml2, Wr2):
    fs1, el1, er1, fs2, el2, er2 = pl.pallas_call(
        _proj_body, out_shape=_proj_outs(),
        grid=(N // BLK,),
        in_specs=[_ROWS, _ROWS, _WFULL, _WSMALL, _WSMALL,
                  _WFULL, _WSMALL, _WSMALL],
        out_specs=_PROJ_OUT_SPECS,
    )(h1a, h1b, W1, Ml1, Wr1, W2, Ml2, Wr2)
    return (fs1.reshape(2 * N, 64), el1.reshape(2 * N, H), er1,
            fs2.reshape(2 * N, 64), el2.reshape(2 * N, H), er2)


def _update(o_ref, bias_ref, h_ref):
    o = jnp.concatenate([o_ref[0], o_ref[1]], axis=1)
    return jnp.maximum(o + bias_ref[...], 0.0) + h_ref[...]


def _upd_proj_body(o1_ref, b1_ref, h1a_ref, o2_ref, b2_ref, h1b_ref,
                   W1_ref, Ml1_ref, Wr1_ref, W2_ref, Ml2_ref, Wr2_ref,
                   h1ao, h1bo, fs1o, el1o, er1o, fs2o, el2o, er2o):
    a = _update(o1_ref, b1_ref, h1a_ref)
    b_ = _update(o2_ref, b2_ref, h1b_ref)
    h1ao[...] = a
    h1bo[...] = b_
    _proj_pair(a, b_, W1_ref[...], Ml1_ref[...], Wr1_ref[...],
               W2_ref[...], Ml2_ref[...], Wr2_ref[...],
               fs1o, el1o, er1o, fs2o, el2o, er2o)


_OBLK = pl.BlockSpec((2, BLK, 64), lambda i: (0, i, 0))


def _tc_upd_proj(o1, b1, h1a, o2, b2, h1b, W1, Ml1, Wr1, W2, Ml2, Wr2):
    outs = [jax.ShapeDtypeStruct((N, HID), _f32),
            jax.ShapeDtypeStruct((N, HID), _f32)] + _proj_outs()
    h1an, h1bn, fs1, el1, er1, fs2, el2, er2 = pl.pallas_call(
        _upd_proj_body, out_shape=outs,
        grid=(N // BLK,),
        in_specs=[_OBLK, _BIAS, _ROWS, _OBLK, _BIAS, _ROWS,
                  _WFULL, _WSMALL, _WSMALL, _WFULL, _WSMALL, _WSMALL],
        out_specs=[_ROWS, _ROWS] + _PROJ_OUT_SPECS,
    )(o1, b1, h1a, o2, b2, h1b, W1, Ml1, Wr1, W2, Ml2, Wr2)
    return (h1an, h1bn, fs1.reshape(2 * N, 64), el1.reshape(2 * N, H),
            er1, fs2.reshape(2 * N, 64), el2.reshape(2 * N, H), er2)


def _dec_body(o1_ref, b1_ref, h1a_ref, o2_ref, b2_ref, h1b_ref,
              W0t_ref, W0b_ref, g_ref, bb_ref, W1_ref, o_ref):
    oa = jnp.concatenate([o1_ref[0, 0:N], o1_ref[1, 0:N]], axis=1)
    a = jnp.maximum(oa + b1_ref[...], 0.0) + h1a_ref[...]
    ob = jnp.concatenate([o2_ref[0, 0:N], o2_ref[1, 0:N]], axis=1)
    b_ = jnp.maximum(ob + b2_ref[...], 0.0) + h1b_ref[...]
    h = (jnp.dot(a, W0t_ref[...], preferred_element_type=_f32)
         + jnp.dot(b_, W0b_ref[...], preferred_element_type=_f32))
    m = jnp.mean(h, axis=0, keepdims=True)
    v = jnp.mean((h - m) ** 2, axis=0, keepdims=True)
    hn = g_ref[...] * (h - m) / jnp.sqrt(v + 1e-5) + bb_ref[...]
    o_ref[...] = jnp.dot(jnp.maximum(hn, 0.0), W1_ref[...],
                         preferred_element_type=_f32)


def _tc_dec(o1, b1, h1a, o2, b2, h1b, W0t, W0b, g, bb, W1):
    return pl.pallas_call(
        _dec_body,
        out_shape=jax.ShapeDtypeStruct((N, HID), _f32),
    )(o1, b1, h1a, o2, b2, h1b, W0t, W0b, g, bb, W1)


def _sc_body(fs_ref, el_ref, er_ref, src3_ref, dst3_ref, out_ref,
             idxf2, idxd0, idxd1, idxe0, idxe1, idxd_f,
             fsbuf, elg0, elg1, erg0, erg1,
             num, den, esem0, esem1, fsem, ssem):
    c = lax.axis_index("c")
    s = lax.axis_index("s")
    r0 = s * ROWS_PT
    idxd = (idxd0, idxd1)
    idxe = (idxe0, idxe1)
    elg = (elg0, elg1)
    erg = (erg0, erg1)
    esem = (esem0, esem1)

    def zb(i, _):
        z = jnp.zeros((16,), _f32)
        for d4 in range(4):
            fsbuf[i, pl.ds(d4 * 16, 16)] = z
        elg0[i] = z
        return 0
    lax.fori_loop(0, CH, zb, 0)
    pltpu.sync_copy(fsbuf.at[pl.ds(0, 512)], num.at[pl.ds(r0, 512)])
    pltpu.sync_copy(fsbuf.at[pl.ds(0, ROWS_PT - 512)],
                    num.at[pl.ds(r0 + 512, ROWS_PT - 512)])
    pltpu.sync_copy(elg0.at[pl.ds(0, 512)], den.at[pl.ds(r0, 512)])
    pltpu.sync_copy(elg0.at[pl.ds(0, ROWS_PT - 512)],
                    den.at[pl.ds(r0 + 512, ROWS_PT - 512)])

    pltpu.sync_copy(src3_ref.at[s], idxf2)

    def offk(k, _):
        def off(j):
            idxf2[k, pl.ds(j * 16, 16)] = (
                idxf2[k, pl.ds(j * 16, 16)] + c * N)
        plsc.parallel_loop(0, CH // 16, 1, unroll=4)(off)
        return 0
    lax.fori_loop(0, NCHUNK, offk, 0)
    plsc.subcore_barrier()

    def stage_elr(k, b):
        pltpu.sync_copy(dst3_ref.at[s, k], idxd[b])

        def clmp(j):
            idxe[b][pl.ds(j * 16, 16)] = jnp.minimum(
                idxd[b][pl.ds(j * 16, 16)], N - 1)
        plsc.parallel_loop(0, CH // 16, 1, unroll=4)(clmp)
        pltpu.async_copy(el_ref.at[idxf2.at[k]], elg[b], esem[b])
        pltpu.async_copy(er_ref.at[idxe[b]], erg[b], esem[b])

    def wait_elr(k, b):
        pltpu.make_async_copy(el_ref.at[idxf2.at[k]], elg[b],
                              esem[b]).wait()
        pltpu.make_async_copy(er_ref.at[idxe[b]], erg[b], esem[b]).wait()

    def half(k, b):
        cp = pltpu.async_copy(fs_ref.at[idxf2.at[k]], fsbuf, fsem)
        kn = jnp.minimum(k + 1, NCHUNK - 1)
        stage_elr(kn, 1 - b)
        cp.wait()

        def edge(i):
            ev = elg[b][i] + erg[b][i]
            ev = jnp.maximum(ev, 0.2 * ev)
            ee = jnp.exp(ev)
            elg[b][i] = ee
            for d4 in range(4):
                fsbuf[i, pl.ds(d4 * 16, 16)] = (
                    fsbuf[i, pl.ds(d4 * 16, 16)] * ee)
        plsc.parallel_loop(0, CH, 1, unroll=4)(edge)

        def cpy(j):
            idxd_f[pl.ds(j * 16, 16)] = idxd[b][pl.ds(j * 16, 16)]
        plsc.parallel_loop(0, CH // 16, 1, unroll=4)(cpy)
        wait_elr(kn, 1 - b)
        sc1 = pltpu.async_copy(fsbuf, num.at[idxd_f], ssem, add=True)
        sc2 = pltpu.async_copy(elg[b], den.at[idxd_f], ssem, add=True)
        sc1.wait()
        sc2.wait()

    stage_elr(0, 0)
    wait_elr(0, 0)

    def pipe(m, _):
        half(2 * m, 0)
        half(2 * m + 1, 1)
        return 0
    lax.fori_loop(0, NCHUNK // 2, pipe, 0)

    plsc.subcore_barrier()

    for off0, nb in ((0, 512), (512, ROWS_PT - 512)):
        pltpu.sync_copy(num.at[pl.ds(r0 + off0, nb)], fsbuf.at[pl.ds(0, nb)])
        pltpu.sync_copy(den.at[pl.ds(r0 + off0, nb)], elg0.at[pl.ds(0, nb)])

        def rowdiv(i):
            dv = elg0[i]
            rv = jnp.where(dv > 0.0, 1.0 / dv, jnp.zeros_like(dv))
            for d4 in range(4):
                fsbuf[i, pl.ds(d4 * 16, 16)] = (
                    fsbuf[i, pl.ds(d4 * 16, 16)] * rv)
        plsc.parallel_loop(0, nb, 1, unroll=4)(rowdiv)
        pltpu.sync_copy(fsbuf.at[pl.ds(0, nb)],
                        out_ref.at[c, pl.ds(r0 + off0, nb)])


@functools.partial(
    pl.kernel,
    out_type=jax.ShapeDtypeStruct((NC, NPAD, 64), _f32),
    mesh=plsc.VectorSubcoreMesh(core_axis_name="c", subcore_axis_name="s"),
    compiler_params=pltpu.CompilerParams(use_tc_tiling_on_sc=False),
    scratch_types=[
        pltpu.VMEM((NCHUNK, CH), jnp.int32),
        pltpu.VMEM((CH,), jnp.int32),
        pltpu.VMEM((CH,), jnp.int32),
        pltpu.VMEM((CH,), jnp.int32),
        pltpu.VMEM((CH,), jnp.int32),
        pltpu.VMEM((CH,), jnp.int32),
        pltpu.VMEM((CH, 64), _f32),
        pltpu.VMEM((CH, H), _f32),
        pltpu.VMEM((CH, H), _f32),
        pltpu.VMEM((CH, H), _f32),
        pltpu.VMEM((CH, H), _f32),
        pltpu.VMEM_SHARED((NPAD, 64), _f32),
        pltpu.VMEM_SHARED((NPAD, H), _f32),
        pltpu.SemaphoreType.DMA,
        pltpu.SemaphoreType.DMA,
        pltpu.SemaphoreType.DMA,
        pltpu.SemaphoreType.DMA,
    ],
)
def _sc_edge(fs_cat, el_cat, er, src3, dst3, out,
             idxf2, idxd0, idxd1, idxe0, idxe1, idxd_f,
             fsbuf, elg0, elg1, erg0, erg1,
             num, den, esem0, esem1, fsem, ssem):
    _sc_body(fs_cat, el_cat, er, src3, dst3, out,
             idxf2, idxd0, idxd1, idxe0, idxe1, idxd_f,
             fsbuf, elg0, elg1, erg0, erg1,
             num, den, esem0, esem1, fsem, ssem)


def _make_m(a):
    rows = np.arange(HID)
    cols = np.tile(np.arange(H), HO)
    return jnp.zeros((HID, H), _f32).at[rows, cols].set(a.T.reshape(-1))


def _prep(params):
    p = {'emb': [], 'gat': []}
    eye = jnp.eye(HID, dtype=_f32)
    for t in range(2):
        e = params['emb'][t]
        p['emb'].append({
            'W0': e['W0'],
            'g': e['g'].reshape(1, HID),
            'b': e['b'].reshape(1, HID),
            'W1': (e['W1'] + eye)[:, PERM],
        })
    for l in range(L):
        lay = []
        for r in range(2):
            q = params['gat'][l][r]
            Wpp = q['W'][PERM][:, PERM]
            lay.append({
                'W': Wpp,
                'Ml': _make_m(q['al']),
                'Wr': jnp.dot(Wpp, _make_m(q['ar'])),
                'bias': q['bias'][PERM].reshape(1, HID),
            })
        p['gat'].append(lay)
    dec = params['dec']
    p['dec'] = {
        'W0t': dec['W0'][:HID][PERM],
        'W0b': dec['W0'][HID:][PERM],
        'g': dec['g'].reshape(1, HID),
        'b': dec['b'].reshape(1, HID),
        'W1': dec['W1'],
    }
    return p


def kernel(x1, x2, edge_index_rel1, edge_index_rel2, params):
    p = _prep(params)
    spad = jnp.zeros((EPAD - E,), jnp.int32)
    dpad = jnp.full((EPAD - E,), N, jnp.int32)
    src1 = jnp.concatenate([edge_index_rel1[0], spad]).reshape(NS, NCHUNK, CH)
    dst1 = jnp.concatenate([edge_index_rel1[1], dpad]).reshape(NS, NCHUNK, CH)
    src2 = jnp.concatenate([edge_index_rel2[0], spad]).reshape(NS, NCHUNK, CH)
    dst2 = jnp.concatenate([edge_index_rel2[1], dpad]).reshape(NS, NCHUNK, CH)

    ea, eb = p['emb'][0], p['emb'][1]
    h1a = _tc_embed(x1, ea['W0'], ea['g'], ea['b'], ea['W1'])
    h1b = _tc_embed(x2, eb['W0'], eb['g'], eb['b'], eb['W1'])

    o1 = o2 = None
    for l in range(L):
        q1, q2 = p['gat'][l][0], p['gat'][l][1]
        if l == 0:
            (fs1, el1, er1, fs2, el2, er2) = _tc_proj0(
                h1a, h1b, q1['W'], q1['Ml'], q1['Wr'],
                q2['W'], q2['Ml'], q2['Wr'])
        else:
            q1p, q2p = p['gat'][l - 1][0], p['gat'][l - 1][1]
            (h1a, h1b, fs1, el1, er1, fs2, el2, er2) = _tc_upd_proj(
                o1, q2p['bias'], h1a, o2, q1p['bias'], h1b,
                q1['W'], q1['Ml'], q1['Wr'],
                q2['W'], q2['Ml'], q2['Wr'])
        o2 = _sc_edge(fs1, el1, er1, src1, dst1)
        o1 = _sc_edge(fs2, el2, er2, src2, dst2)

    q1p, q2p = p['gat'][L - 1][0], p['gat'][L - 1][1]
    d = p['dec']
    return _tc_dec(o1, q2p['bias'], h1a, o2, q1p['bias'], h1b,
                   d['W0t'], d['W0b'], d['g'], d['b'], d['W1'])


# speedup vs baseline: 1.0298x; 1.0298x over previous
import functools

import numpy as np
import jax
import jax.numpy as jnp
from jax import lax
from jax.experimental import pallas as pl
from jax.experimental.pallas import tpu as pltpu
from jax.experimental.pallas import tpu_sc as plsc

N = 10000
E = 160000
HID = 128
H = 16
HO = 8
L = 2

NC = 2
NS = 16
CH = 512
EPT = 10240
EPAD = NS * EPT
NCHUNK = EPT // CH
ROWS_PT = 632
NPAD = NS * ROWS_PT

PERM = np.array([(j % H) * HO + j // H for j in range(HID)], np.int32)

_f32 = jnp.float32


def _embed_body(x_ref, w0_ref, g_ref, b_ref, w1_ref, o_ref):
    h = jnp.dot(x_ref[...], w0_ref[...], preferred_element_type=_f32)
    m = jnp.mean(h, axis=0, keepdims=True)
    v = jnp.mean((h - m) ** 2, axis=0, keepdims=True)
    hn = g_ref[...] * (h - m) / jnp.sqrt(v + 1e-5) + b_ref[...]
    o_ref[...] = jnp.dot(jnp.maximum(hn, 0.0), w1_ref[...],
                         preferred_element_type=_f32)


def _tc_embed(x, w0, g, b, w1):
    return pl.pallas_call(
        _embed_body,
        out_shape=jax.ShapeDtypeStruct((N, HID), _f32),
    )(x, w0, g, b, w1)


BLK = 2000


def _proj_pair(a, b_, W1, Ml1, Wr1, W2, Ml2, Wr2,
               fs1o, el1o, er1o, fs2o, el2o, er2o):
    fs1 = jnp.dot(a, W1, preferred_element_type=_f32)
    fs1o[0] = fs1[:, 0:64]
    fs1o[1] = fs1[:, 64:128]
    el1o[...] = jnp.dot(fs1, Ml1, preferred_element_type=_f32)
    er1o[...] = jnp.dot(b_, Wr1, preferred_element_type=_f32)
    fs2 = jnp.dot(b_, W2, preferred_element_type=_f32)
    fs2o[0] = fs2[:, 0:64]
    fs2o[1] = fs2[:, 64:128]
    el2o[...] = jnp.dot(fs2, Ml2, preferred_element_type=_f32)
    er2o[...] = jnp.dot(a, Wr2, preferred_element_type=_f32)


def _proj_body(h1a_ref, h1b_ref, W1_ref, Ml1_ref, Wr1_ref,
               W2_ref, Ml2_ref, Wr2_ref,
               fs1o, el1o, er1o, fs2o, el2o, er2o):
    _proj_pair(h1a_ref[...], h1b_ref[...], W1_ref[...], Ml1_ref[...],
               Wr1_ref[...], W2_ref[...], Ml2_ref[...], Wr2_ref[...],
               fs1o, el1o, er1o, fs2o, el2o, er2o)


_ROWS = pl.BlockSpec((BLK, HID), lambda i: (i, 0))
_WFULL = pl.BlockSpec((HID, HID), lambda i: (0, 0))
_WSMALL = pl.BlockSpec((HID, H), lambda i: (0, 0))
_BIAS = pl.BlockSpec((1, HID), lambda i: (0, 0))
_FSBLK = pl.BlockSpec((2, BLK, 64), lambda i: (0, i, 0))
_ELBLK = pl.BlockSpec((BLK, H), lambda i: (i, 0))


def _proj_outs():
    return [
        jax.ShapeDtypeStruct((2, N, 64), _f32),
        jax.ShapeDtypeStruct((N, H), _f32),
        jax.ShapeDtypeStruct((N, H), _f32),
        jax.ShapeDtypeStruct((2, N, 64), _f32),
        jax.ShapeDtypeStruct((N, H), _f32),
        jax.ShapeDtypeStruct((N, H), _f32),
    ]


_PROJ_OUT_SPECS = [_FSBLK, _ELBLK, _ELBLK, _FSBLK, _ELBLK, _ELBLK]


def _tc_proj0(h1a, h1b, W1, Ml1, Wr1, W2, Ml2, Wr2):
    fs1, el1, er1, fs2, el2, er2 = pl.pallas_call(
        _proj_body, out_shape=_proj_outs(),
        grid=(N // BLK,),
        in_specs=[_ROWS, _ROWS, _WFULL, _WSMALL, _WSMALL,
                  _WFULL, _WSMALL, _WSMALL],
        out_specs=_PROJ_OUT_SPECS,
    )(h1a, h1b, W1, Ml1, Wr1, W2, Ml2, Wr2)
    return (fs1.reshape(2 * N, 64), el1, er1,
            fs2.reshape(2 * N, 64), el2, er2)


def _update(o_ref, bias_ref, h_ref):
    o = jnp.concatenate([o_ref[0], o_ref[1]], axis=1)
    return jnp.maximum(o + bias_ref[...], 0.0) + h_ref[...]


def _upd_proj_body(o1_ref, b1_ref, h1a_ref, o2_ref, b2_ref, h1b_ref,
                   W1_ref, Ml1_ref, Wr1_ref, W2_ref, Ml2_ref, Wr2_ref,
                   h1ao, h1bo, fs1o, el1o, er1o, fs2o, el2o, er2o):
    a = _update(o1_ref, b1_ref, h1a_ref)
    b_ = _update(o2_ref, b2_ref, h1b_ref)
    h1ao[...] = a
    h1bo[...] = b_
    _proj_pair(a, b_, W1_ref[...], Ml1_ref[...], Wr1_ref[...],
               W2_ref[...], Ml2_ref[...], Wr2_ref[...],
               fs1o, el1o, er1o, fs2o, el2o, er2o)


_OBLK = pl.BlockSpec((2, BLK, 64), lambda i: (0, i, 0))


def _tc_upd_proj(o1, b1, h1a, o2, b2, h1b, W1, Ml1, Wr1, W2, Ml2, Wr2):
    outs = [jax.ShapeDtypeStruct((N, HID), _f32),
            jax.ShapeDtypeStruct((N, HID), _f32)] + _proj_outs()
    h1an, h1bn, fs1, el1, er1, fs2, el2, er2 = pl.pallas_call(
        _upd_proj_body, out_shape=outs,
        grid=(N // BLK,),
        in_specs=[_OBLK, _BIAS, _ROWS, _OBLK, _BIAS, _ROWS,
                  _WFULL, _WSMALL, _WSMALL, _WFULL, _WSMALL, _WSMALL],
        out_specs=[_ROWS, _ROWS] + _PROJ_OUT_SPECS,
    )(o1, b1, h1a, o2, b2, h1b, W1, Ml1, Wr1, W2, Ml2, Wr2)
    return (h1an, h1bn, fs1.reshape(2 * N, 64), el1, er1,
            fs2.reshape(2 * N, 64), el2, er2)


def _dec_body(o1_ref, b1_ref, h1a_ref, o2_ref, b2_ref, h1b_ref,
              W0t_ref, W0b_ref, g_ref, bb_ref, W1_ref, o_ref):
    oa = jnp.concatenate([o1_ref[0, 0:N], o1_ref[1, 0:N]], axis=1)
    a = jnp.maximum(oa + b1_ref[...], 0.0) + h1a_ref[...]
    ob = jnp.concatenate([o2_ref[0, 0:N], o2_ref[1, 0:N]], axis=1)
    b_ = jnp.maximum(ob + b2_ref[...], 0.0) + h1b_ref[...]
    h = (jnp.dot(a, W0t_ref[...], preferred_element_type=_f32)
         + jnp.dot(b_, W0b_ref[...], preferred_element_type=_f32))
    m = jnp.mean(h, axis=0, keepdims=True)
    v = jnp.mean((h - m) ** 2, axis=0, keepdims=True)
    hn = g_ref[...] * (h - m) / jnp.sqrt(v + 1e-5) + bb_ref[...]
    o_ref[...] = jnp.dot(jnp.maximum(hn, 0.0), W1_ref[...],
                         preferred_element_type=_f32)


def _tc_dec(o1, b1, h1a, o2, b2, h1b, W0t, W0b, g, bb, W1):
    return pl.pallas_call(
        _dec_body,
        out_shape=jax.ShapeDtypeStruct((N, HID), _f32),
    )(o1, b1, h1a, o2, b2, h1b, W0t, W0b, g, bb, W1)


def _sc_body(fs_ref, el_ref, er_ref, src_ref, dst_ref, out_ref,
             idxs0, idxs1, idxd0, idxd1, idxf0, idxf1, idxd_f,
             fsbuf, elg0, elg1, erg0, erg1,
             num, den, esem0, esem1, fsem, ssem):
    c = lax.axis_index("c")
    s = lax.axis_index("s")
    r0 = s * ROWS_PT
    idxs = (idxs0, idxs1)
    idxd = (idxd0, idxd1)
    idxf = (idxf0, idxf1)
    elg = (elg0, elg1)
    erg = (erg0, erg1)
    esem = (esem0, esem1)

    def zb(i, _):
        z = jnp.zeros((16,), _f32)
        for d4 in range(4):
            fsbuf[i, pl.ds(d4 * 16, 16)] = z
        elg0[i] = z
        return 0
    lax.fori_loop(0, CH, zb, 0)
    pltpu.sync_copy(fsbuf.at[pl.ds(0, 512)], num.at[pl.ds(r0, 512)])
    pltpu.sync_copy(fsbuf.at[pl.ds(0, ROWS_PT - 512)],
                    num.at[pl.ds(r0 + 512, ROWS_PT - 512)])
    pltpu.sync_copy(elg0.at[pl.ds(0, 512)], den.at[pl.ds(r0, 512)])
    pltpu.sync_copy(elg0.at[pl.ds(0, ROWS_PT - 512)],
                    den.at[pl.ds(r0 + 512, ROWS_PT - 512)])
    plsc.subcore_barrier()

    def stage_idx_and_elr(k, b):
        eb = s * EPT + k * CH
        pltpu.sync_copy(src_ref.at[pl.ds(eb, CH)], idxs[b])
        pltpu.sync_copy(dst_ref.at[pl.ds(eb, CH)], idxd[b])

        def off(j):
            idxf[b][pl.ds(j * 16, 16)] = idxs[b][pl.ds(j * 16, 16)] + c * N
        plsc.parallel_loop(0, CH // 16, 1, unroll=4)(off)
        pltpu.async_copy(el_ref.at[idxs[b]], elg[b], esem[b])
        pltpu.async_copy(er_ref.at[idxd[b]], erg[b], esem[b])

    def wait_elr(b):
        pltpu.make_async_copy(el_ref.at[idxs[b]], elg[b], esem[b]).wait()
        pltpu.make_async_copy(er_ref.at[idxd[b]], erg[b], esem[b]).wait()

    def half(k, b):
        cp = pltpu.async_copy(fs_ref.at[idxf[b]], fsbuf, fsem)
        kn = jnp.minimum(k + 1, NCHUNK - 1)
        stage_idx_and_elr(kn, 1 - b)
        cp.wait()

        def edge(i):
            ev = elg[b][i] + erg[b][i]
            ev = jnp.maximum(ev, 0.2 * ev)
            ee = jnp.exp(ev)
            elg[b][i] = ee
            for d4 in range(4):
                fsbuf[i, pl.ds(d4 * 16, 16)] = (
                    fsbuf[i, pl.ds(d4 * 16, 16)] * ee)
        plsc.parallel_loop(0, CH, 1, unroll=4)(edge)

        def cpy(j):
            idxd_f[pl.ds(j * 16, 16)] = idxd[b][pl.ds(j * 16, 16)]
        plsc.parallel_loop(0, CH // 16, 1, unroll=4)(cpy)
        wait_elr(1 - b)
        sc1 = pltpu.async_copy(fsbuf, num.at[idxd_f], ssem, add=True)
        sc2 = pltpu.async_copy(elg[b], den.at[idxd_f], ssem, add=True)
        sc1.wait()
        sc2.wait()

    stage_idx_and_elr(0, 0)
    wait_elr(0)

    def pipe(m, _):
        half(2 * m, 0)
        half(2 * m + 1, 1)
        return 0
    lax.fori_loop(0, NCHUNK // 2, pipe, 0)

    plsc.subcore_barrier()

    for off0, nb in ((0, 512), (512, ROWS_PT - 512)):
        pltpu.sync_copy(num.at[pl.ds(r0 + off0, nb)], fsbuf.at[pl.ds(0, nb)])
        pltpu.sync_copy(den.at[pl.ds(r0 + off0, nb)], elg0.at[pl.ds(0, nb)])

        def rowdiv(i):
            dv = elg0[i]
            rv = jnp.where(dv > 0.0, 1.0 / dv, jnp.zeros_like(dv))
            for d4 in range(4):
                fsbuf[i, pl.ds(d4 * 16, 16)] = (
                    fsbuf[i, pl.ds(d4 * 16, 16)] * rv)
        plsc.parallel_loop(0, nb, 1, unroll=4)(rowdiv)
        pltpu.sync_copy(fsbuf.at[pl.ds(0, nb)],
                        out_ref.at[c, pl.ds(r0 + off0, nb)])


@functools.partial(
    pl.kernel,
    out_type=jax.ShapeDtypeStruct((NC, NPAD, 64), _f32),
    mesh=plsc.VectorSubcoreMesh(core_axis_name="c", subcore_axis_name="s"),
    compiler_params=pltpu.CompilerParams(use_tc_tiling_on_sc=False),
    scratch_types=(
        [pltpu.VMEM((CH,), jnp.int32)] * 7
        + [pltpu.VMEM((CH, 64), _f32),
           pltpu.VMEM((CH, H), _f32), pltpu.VMEM((CH, H), _f32),
           pltpu.VMEM((CH, H), _f32), pltpu.VMEM((CH, H), _f32),
           pltpu.VMEM_SHARED((NPAD, 64), _f32),
           pltpu.VMEM_SHARED((NPAD, H), _f32),
           pltpu.SemaphoreType.DMA,
           pltpu.SemaphoreType.DMA,
           pltpu.SemaphoreType.DMA,
           pltpu.SemaphoreType.DMA]
    ),
)
def _sc_edge(fs_cat, el, er, srcp, dstp, out,
             idxs0, idxs1, idxd0, idxd1, idxf0, idxf1, idxd_f,
             fsbuf, elg0, elg1, erg0, erg1,
             num, den, esem0, esem1, fsem, ssem):
    _sc_body(fs_cat, el, er, srcp, dstp, out,
             idxs0, idxs1, idxd0, idxd1, idxf0, idxf1, idxd_f,
             fsbuf, elg0, elg1, erg0, erg1,
             num, den, esem0, esem1, fsem, ssem)


def _make_m(a):
    rows = np.arange(HID)
    cols = np.tile(np.arange(H), HO)
    return jnp.zeros((HID, H), _f32).at[rows, cols].set(a.T.reshape(-1))


def _prep(params):
    p = {'emb': [], 'gat': []}
    eye = jnp.eye(HID, dtype=_f32)
    for t in range(2):
        e = params['emb'][t]
        p['emb'].append({
            'W0': e['W0'],
            'g': e['g'].reshape(1, HID),
            'b': e['b'].reshape(1, HID),
            'W1': (e['W1'] + eye)[:, PERM],
        })
    for l in range(L):
        lay = []
        for r in range(2):
            q = params['gat'][l][r]
            Wpp = q['W'][PERM][:, PERM]
            lay.append({
                'W': Wpp,
                'Ml': _make_m(q['al']),
                'Wr': jnp.dot(Wpp, _make_m(q['ar'])),
                'bias': q['bias'][PERM].reshape(1, HID),
            })
        p['gat'].append(lay)
    dec = params['dec']
    p['dec'] = {
        'W0t': dec['W0'][:HID][PERM],
        'W0b': dec['W0'][HID:][PERM],
        'g': dec['g'].reshape(1, HID),
        'b': dec['b'].reshape(1, HID),
        'W1': dec['W1'],
    }
    return p


def kernel(x1, x2, edge_index_rel1, edge_index_rel2, params):
    p = _prep(params)
    spad = jnp.zeros((EPAD - E,), jnp.int32)
    dpad = jnp.full((EPAD - E,), N, jnp.int32)
    src1 = jnp.concatenate([edge_index_rel1[0], spad])
    dst1 = jnp.concatenate([edge_index_rel1[1], dpad])
    src2 = jnp.concatenate([edge_index_rel2[0], spad])
    dst2 = jnp.concatenate([edge_index_rel2[1], dpad])

    ea, eb = p['emb'][0], p['emb'][1]
    h1a = _tc_embed(x1, ea['W0'], ea['g'], ea['b'], ea['W1'])
    h1b = _tc_embed(x2, eb['W0'], eb['g'], eb['b'], eb['W1'])

    o1 = o2 = None
    for l in range(L):
        q1, q2 = p['gat'][l][0], p['gat'][l][1]
        if l == 0:
            (fs1, el1, er1, fs2, el2, er2) = _tc_proj0(
                h1a, h1b, q1['W'], q1['Ml'], q1['Wr'],
                q2['W'], q2['Ml'], q2['Wr'])
        else:
            q1p, q2p = p['gat'][l - 1][0], p['gat'][l - 1][1]
            (h1a, h1b, fs1, el1, er1, fs2, el2, er2) = _tc_upd_proj(
                o1, q2p['bias'], h1a, o2, q1p['bias'], h1b,
                q1['W'], q1['Ml'], q1['Wr'],
                q2['W'], q2['Ml'], q2['Wr'])
        o2 = _sc_edge(fs1, el1, er1, src1, dst1)
        o1 = _sc_edge(fs2, el2, er2, src2, dst2)

    q1p, q2p = p['gat'][L - 1][0], p['gat'][L - 1][1]
    d = p['dec']
    return _tc_dec(o1, q2p['bias'], h1a, o2, q1p['bias'], h1b,
                   d['W0t'], d['W0b'], d['g'], d['b'], d['W1'])
